# baseline (speedup 1.0000x reference)
import dataclasses
import functools

import jax
import jax.numpy as jnp
from jax import lax
from jax.experimental import pallas as pl
from jax.experimental.pallas import tpu as pltpu
from jax.experimental.pallas import tpu_sc as plsc

N = 10000
D = 128
E = 320000

NS = 16
CHUNK = 128
E_PAD = 327680
NCHUNK_ALL = E_PAD // CHUNK
NCHUNK_SUB = NCHUNK_ALL // NS
RP = 632
N_PAD = NS * RP
HR = N_PAD // 16

ROW_BLK = 632
GRID = N_PAD // ROW_BLK

_mesh = plsc.VectorSubcoreMesh(core_axis_name="c", subcore_axis_name="s")
_CP = dataclasses.replace(pltpu.CompilerParams(), needs_layout_passes=False)


def _i16():
    return lax.iota(jnp.int32, 16)


@jax.jit
def _hist(dst2):
    @functools.partial(
        pl.kernel,
        out_type=jax.ShapeDtypeStruct((2, NS, HR, 16), jnp.float32),
        mesh=_mesh,
        compiler_params=_CP,
        scratch_types=[
            pltpu.VMEM((8, 16), jnp.int32),
            pltpu.VMEM((HR, 16), jnp.float32),
        ],
    )
    def k(dst_hbm, out_hbm, idx_v, hist_v):
        cid = lax.axis_index("c")
        sid = lax.axis_index("s")
        ones = jnp.full((16,), 1.0, jnp.float32)

        @pl.loop(0, HR)
        def _(i):
            hist_v[i, :] = jnp.zeros((16,), jnp.float32)

        base = sid * (NCHUNK_SUB * 8)

        @pl.loop(0, NCHUNK_SUB)
        def _(cnk):
            pltpu.sync_copy(dst_hbm.at[cid, 0, pl.ds(base + cnk * 8, 8), :],
                            idx_v)
            for j in range(8):
                dv = idx_v[j, :]
                plsc.addupdate_scatter(hist_v, [dv // 16, dv % 16], ones)

        pltpu.sync_copy(hist_v, out_hbm.at[cid, sid])

    return k(dst2)


@jax.jit
def _edge_pass(y_t, y_c, src2, dst2):
    @functools.partial(
        pl.kernel,
        out_type=jax.ShapeDtypeStruct((2, NS, RP, D), jnp.float32),
        mesh=_mesh,
        compiler_params=_CP,
        scratch_types=[
            pltpu.VMEM((CHUNK,), jnp.int32),
            pltpu.VMEM((8, 16), jnp.int32),
            pltpu.VMEM((CHUNK, D), jnp.float32),
            pltpu.VMEM((RP, D), jnp.float32),
            pltpu.SemaphoreType.DMA,
        ],
    )
    def k(yt_hbm, yc_hbm, src_hbm, dst_hbm, out_hbm,
          src_v, dst_v, rows_v, acc_v, sem):
        cid = lax.axis_index("c")
        sid = lax.axis_index("s")
        lo = jnp.broadcast_to(sid * RP, (16,)).astype(jnp.int32)

        @pl.loop(0, RP)
        def _(i):
            for g in range(8):
                acc_v[i, pl.ds(g * 16, 16)] = jnp.zeros((16,), jnp.float32)

        def scan(y_hbm):
            @pl.loop(0, NCHUNK_ALL)
            def _(cnk):
                pltpu.sync_copy(src_hbm.at[cid, 0, pl.ds(cnk * CHUNK, CHUNK)],
                                src_v)
                pltpu.sync_copy(dst_hbm.at[cid, 0, pl.ds(cnk * 8, 8), :],
                                dst_v)
                pltpu.async_copy(y_hbm.at[src_v], rows_v, sem).wait()
                for j in range(8):
                    dv = dst_v[j, :] - lo
                    m32 = jnp.where(
                        jnp.logical_and(dv >= 0, dv < RP), 1, 0)
                    dvc = jnp.clip(dv, 0, RP - 1)
                    for kk in range(16):
                        sm = jnp.sum(jnp.where(_i16() == kk, m32, 0))

                        @pl.when(sm > 0)
                        def _():
                            row = jnp.take(dvc, jnp.full((16,), kk, jnp.int32))
                            for g in range(8):
                                plsc.addupdate_scatter(
                                    acc_v, [row, g * 16 + _i16()],
                                    rows_v[j * 16 + kk, pl.ds(g * 16, 16)])

        @pl.when(cid == 0)
        def _():
            scan(yt_hbm)

        @pl.when(cid == 1)
        def _():
            scan(yc_hbm)

        pltpu.sync_copy(acc_v, out_hbm.at[cid, sid])

    return k(y_t, y_c, src2, dst2)


def _matmul_body(x_ref, wt_ref, wc_ref, ot_ref, oc_ref):
    x = x_ref[...]
    ot_ref[...] = jnp.dot(x, wt_ref[...], preferred_element_type=jnp.float32)
    oc_ref[...] = jnp.dot(x, wc_ref[...], preferred_element_type=jnp.float32)


@jax.jit
def _matmul(x, W_t, W_c):
    return pl.pallas_call(
        _matmul_body,
        grid=(GRID,),
        in_specs=[
            pl.BlockSpec((ROW_BLK, D), lambda i: (i, 0)),
            pl.BlockSpec((D, D), lambda i: (0, 0)),
            pl.BlockSpec((D, D), lambda i: (0, 0)),
        ],
        out_specs=[
            pl.BlockSpec((ROW_BLK, D), lambda i: (i, 0)),
            pl.BlockSpec((ROW_BLK, D), lambda i: (i, 0)),
        ],
        out_shape=[
            jax.ShapeDtypeStruct((N_PAD, D), jnp.float32),
            jax.ShapeDtypeStruct((N_PAD, D), jnp.float32),
        ],
    )(x, W_t, W_c)


def _prescale_body(xwt_ref, xwc_ref, hist_ref, yt_ref, yc_ref):
    deg_t = 1.0 + jnp.sum(hist_ref[0], axis=1)[:, None]
    deg_c = 1.0 + jnp.sum(hist_ref[1], axis=1)[:, None]
    yt_ref[...] = xwt_ref[...] * lax.rsqrt(deg_t)
    yc_ref[...] = xwc_ref[...] * lax.rsqrt(deg_c)


@jax.jit
def _prescale(xw_t, xw_c, hist_flat):
    return pl.pallas_call(
        _prescale_body,
        grid=(GRID,),
        in_specs=[
            pl.BlockSpec((ROW_BLK, D), lambda i: (i, 0)),
            pl.BlockSpec((ROW_BLK, D), lambda i: (i, 0)),
            pl.BlockSpec((2, ROW_BLK, NS), lambda i: (0, i, 0)),
        ],
        out_specs=[
            pl.BlockSpec((ROW_BLK, D), lambda i: (i, 0)),
            pl.BlockSpec((ROW_BLK, D), lambda i: (i, 0)),
        ],
        out_shape=[
            jax.ShapeDtypeStruct((N_PAD, D), jnp.float32),
            jax.ShapeDtypeStruct((N_PAD, D), jnp.float32),
        ],
    )(xw_t, xw_c, hist_flat)


def _final_body(acc_ref, yt_ref, yc_ref, hist_ref, bt_ref, bc_ref, o_ref):
    dis_t = lax.rsqrt(1.0 + jnp.sum(hist_ref[0], axis=1)[:, None])
    dis_c = lax.rsqrt(1.0 + jnp.sum(hist_ref[1], axis=1)[:, None])
    o_ref[...] = (
        (acc_ref[0, 0] + yt_ref[...]) * dis_t
        + (acc_ref[1, 0] + yc_ref[...]) * dis_c
        + bt_ref[...] + bc_ref[...]
    )


@jax.jit
def _final(acc, y_t, y_c, hist_flat, b_t, b_c):
    return pl.pallas_call(
        _final_body,
        grid=(GRID,),
        in_specs=[
            pl.BlockSpec((2, 1, ROW_BLK, D), lambda i: (0, i, 0, 0)),
            pl.BlockSpec((ROW_BLK, D), lambda i: (i, 0)),
            pl.BlockSpec((ROW_BLK, D), lambda i: (i, 0)),
            pl.BlockSpec((2, ROW_BLK, NS), lambda i: (0, i, 0)),
            pl.BlockSpec((1, D), lambda i: (0, 0)),
            pl.BlockSpec((1, D), lambda i: (0, 0)),
        ],
        out_specs=pl.BlockSpec((ROW_BLK, D), lambda i: (i, 0)),
        out_shape=jax.ShapeDtypeStruct((N_PAD, D), jnp.float32),
    )(acc, y_t, y_c, hist_flat, b_t, b_c)


def kernel(x, edge_index_tree, edge_index_cooccurs, W_tree, b_tree,
           W_cooccurs, b_cooccurs):
    pad0 = jnp.zeros((E_PAD - E,), jnp.int32)
    padj = jnp.full((E_PAD - E,), N, jnp.int32)
    ei_t = edge_index_tree.astype(jnp.int32)
    ei_c = edge_index_cooccurs.astype(jnp.int32)
    src2 = jnp.stack([jnp.concatenate([ei_t[0], pad0]),
                      jnp.concatenate([ei_c[0], pad0])]).reshape(2, 1, E_PAD)
    dst2 = jnp.stack([jnp.concatenate([ei_t[1], padj]),
                      jnp.concatenate([ei_c[1], padj])]
                     ).reshape(2, 1, E_PAD // 16, 16)

    x_pad = jnp.pad(x, ((0, N_PAD - N), (0, 0)))

    hist = _hist(dst2)
    hist_flat = hist.reshape(2, NS, N_PAD).transpose(0, 2, 1)
    xw_t, xw_c = _matmul(x_pad, W_tree, W_cooccurs)
    y_t, y_c = _prescale(xw_t, xw_c, hist_flat)
    acc = _edge_pass(y_t, y_c, src2, dst2)
    out = _final(acc, y_t, y_c, hist_flat,
                 b_tree.reshape(1, D), b_cooccurs.reshape(1, D))
    return out[:N]


# speedup vs baseline: 1.0916x; 1.0916x over previous
import dataclasses
import functools

import jax
import jax.numpy as jnp
from jax import lax
from jax.experimental import pallas as pl
from jax.experimental.pallas import tpu as pltpu
from jax.experimental.pallas import tpu_sc as plsc

N = 10000
D = 128
E = 320000

NS = 16
CHUNK = 128
E_PAD = 327680
NCHUNK_ALL = E_PAD // CHUNK
NCHUNK_SUB = NCHUNK_ALL // NS
RP = 632
N_PAD = NS * RP
HR = N_PAD // 16

ROW_BLK = 632
GRID = N_PAD // ROW_BLK

_mesh = plsc.VectorSubcoreMesh(core_axis_name="c", subcore_axis_name="s")
_CP = dataclasses.replace(pltpu.CompilerParams(), needs_layout_passes=False)


def _i16():
    return lax.iota(jnp.int32, 16)


@jax.jit
def _hist(dst2):
    @functools.partial(
        pl.kernel,
        out_type=jax.ShapeDtypeStruct((2, NS, HR, 16), jnp.float32),
        mesh=_mesh,
        compiler_params=_CP,
        scratch_types=[
            pltpu.VMEM((8, 16), jnp.int32),
            pltpu.VMEM((HR, 16), jnp.float32),
        ],
    )
    def k(dst_hbm, out_hbm, idx_v, hist_v):
        cid = lax.axis_index("c")
        sid = lax.axis_index("s")
        ones = jnp.full((16,), 1.0, jnp.float32)

        @pl.loop(0, HR)
        def _(i):
            hist_v[i, :] = jnp.zeros((16,), jnp.float32)

        base = sid * (NCHUNK_SUB * 8)

        @pl.loop(0, NCHUNK_SUB)
        def _(cnk):
            pltpu.sync_copy(dst_hbm.at[cid, 0, pl.ds(base + cnk * 8, 8), :],
                            idx_v)
            for j in range(8):
                dv = idx_v[j, :]
                plsc.addupdate_scatter(hist_v, [dv // 16, dv % 16], ones)

        pltpu.sync_copy(hist_v, out_hbm.at[cid, sid])

    return k(dst2)


@jax.jit
def _edge_pass(y_t, y_c, src2, dst2):
    @functools.partial(
        pl.kernel,
        out_type=jax.ShapeDtypeStruct((2, NS, RP, D), jnp.float32),
        mesh=_mesh,
        compiler_params=_CP,
        scratch_types=[
            pltpu.VMEM((CHUNK,), jnp.int32),
            pltpu.VMEM((8, 16), jnp.int32),
            pltpu.VMEM((CHUNK, D), jnp.float32),
            pltpu.VMEM((RP, D), jnp.float32),
            pltpu.SemaphoreType.DMA,
        ],
    )
    def k(yt_hbm, yc_hbm, src_hbm, dst_hbm, out_hbm,
          src0_v, dst_v, rows0_v, acc_v, sem):
        cid = lax.axis_index("c")
        sid = lax.axis_index("s")
        lo = jnp.broadcast_to(sid * RP, (16,)).astype(jnp.int32)

        @pl.loop(0, RP)
        def _(i):
            for g in range(8):
                acc_v[i, pl.ds(g * 16, 16)] = jnp.zeros((16,), jnp.float32)

        def scan(y_hbm):
            @pl.loop(0, NCHUNK_ALL)
            def _(cnk):
                pltpu.sync_copy(src_hbm.at[cid, 0, cnk, :], src0_v)
                desc = pltpu.async_copy(y_hbm.at[src0_v], rows0_v, sem)
                pltpu.sync_copy(dst_hbm.at[cid, 0, pl.ds(cnk * 8, 8), :],
                                dst_v)
                desc.wait()
                rows_v = rows0_v
                for j in range(8):
                    dv = dst_v[j, :] - lo
                    m32 = jnp.where(
                        jnp.logical_and(dv >= 0, dv < RP), 1, 0)
                    dvc = jnp.clip(dv, 0, RP - 1)
                    gsum = jnp.sum(m32)

                    @pl.when(gsum > 0)
                    def _():
                        for kk in range(16):
                            sm = jnp.sum(jnp.where(_i16() == kk, m32, 0))

                            @pl.when(sm > 0)
                            def _():
                                row = jnp.take(
                                    dvc, jnp.full((16,), kk, jnp.int32))
                                for g in range(8):
                                    plsc.addupdate_scatter(
                                        acc_v, [row, g * 16 + _i16()],
                                        rows_v[j * 16 + kk,
                                               pl.ds(g * 16, 16)])

        @pl.when(cid == 0)
        def _():
            scan(yt_hbm)

        @pl.when(cid == 1)
        def _():
            scan(yc_hbm)

        pltpu.sync_copy(acc_v, out_hbm.at[cid, sid])

    return k(y_t, y_c, src2, dst2)


def _matmul_body(x_ref, wt_ref, wc_ref, ot_ref, oc_ref):
    x = x_ref[...]
    ot_ref[...] = jnp.dot(x, wt_ref[...], preferred_element_type=jnp.float32)
    oc_ref[...] = jnp.dot(x, wc_ref[...], preferred_element_type=jnp.float32)


@jax.jit
def _matmul(x, W_t, W_c):
    return pl.pallas_call(
        _matmul_body,
        grid=(GRID,),
        in_specs=[
            pl.BlockSpec((ROW_BLK, D), lambda i: (i, 0)),
            pl.BlockSpec((D, D), lambda i: (0, 0)),
            pl.BlockSpec((D, D), lambda i: (0, 0)),
        ],
        out_specs=[
            pl.BlockSpec((ROW_BLK, D), lambda i: (i, 0)),
            pl.BlockSpec((ROW_BLK, D), lambda i: (i, 0)),
        ],
        out_shape=[
            jax.ShapeDtypeStruct((N_PAD, D), jnp.float32),
            jax.ShapeDtypeStruct((N_PAD, D), jnp.float32),
        ],
    )(x, W_t, W_c)


def _prescale_body(xwt_ref, xwc_ref, hist_ref, yt_ref, yc_ref):
    deg_t = 1.0 + jnp.sum(hist_ref[0], axis=1)[:, None]
    deg_c = 1.0 + jnp.sum(hist_ref[1], axis=1)[:, None]
    yt_ref[...] = xwt_ref[...] * lax.rsqrt(deg_t)
    yc_ref[...] = xwc_ref[...] * lax.rsqrt(deg_c)


@jax.jit
def _prescale(xw_t, xw_c, hist_flat):
    return pl.pallas_call(
        _prescale_body,
        grid=(GRID,),
        in_specs=[
            pl.BlockSpec((ROW_BLK, D), lambda i: (i, 0)),
            pl.BlockSpec((ROW_BLK, D), lambda i: (i, 0)),
            pl.BlockSpec((2, ROW_BLK, NS), lambda i: (0, i, 0)),
        ],
        out_specs=[
            pl.BlockSpec((ROW_BLK, D), lambda i: (i, 0)),
            pl.BlockSpec((ROW_BLK, D), lambda i: (i, 0)),
        ],
        out_shape=[
            jax.ShapeDtypeStruct((N_PAD, D), jnp.float32),
            jax.ShapeDtypeStruct((N_PAD, D), jnp.float32),
        ],
    )(xw_t, xw_c, hist_flat)


def _final_body(acc_ref, yt_ref, yc_ref, hist_ref, bt_ref, bc_ref, o_ref):
    dis_t = lax.rsqrt(1.0 + jnp.sum(hist_ref[0], axis=1)[:, None])
    dis_c = lax.rsqrt(1.0 + jnp.sum(hist_ref[1], axis=1)[:, None])
    o_ref[...] = (
        (acc_ref[0, 0] + yt_ref[...]) * dis_t
        + (acc_ref[1, 0] + yc_ref[...]) * dis_c
        + bt_ref[...] + bc_ref[...]
    )


@jax.jit
def _final(acc, y_t, y_c, hist_flat, b_t, b_c):
    return pl.pallas_call(
        _final_body,
        grid=(GRID,),
        in_specs=[
            pl.BlockSpec((2, 1, ROW_BLK, D), lambda i: (0, i, 0, 0)),
            pl.BlockSpec((ROW_BLK, D), lambda i: (i, 0)),
            pl.BlockSpec((ROW_BLK, D), lambda i: (i, 0)),
            pl.BlockSpec((2, ROW_BLK, NS), lambda i: (0, i, 0)),
            pl.BlockSpec((1, D), lambda i: (0, 0)),
            pl.BlockSpec((1, D), lambda i: (0, 0)),
        ],
        out_specs=pl.BlockSpec((ROW_BLK, D), lambda i: (i, 0)),
        out_shape=jax.ShapeDtypeStruct((N_PAD, D), jnp.float32),
    )(acc, y_t, y_c, hist_flat, b_t, b_c)


def kernel(x, edge_index_tree, edge_index_cooccurs, W_tree, b_tree,
           W_cooccurs, b_cooccurs):
    pad0 = jnp.zeros((E_PAD - E,), jnp.int32)
    padj = jnp.full((E_PAD - E,), N, jnp.int32)
    ei_t = edge_index_tree.astype(jnp.int32)
    ei_c = edge_index_cooccurs.astype(jnp.int32)
    src2 = jnp.stack([jnp.concatenate([ei_t[0], pad0]),
                      jnp.concatenate([ei_c[0], pad0])]
                     ).reshape(2, 1, E_PAD // CHUNK, CHUNK)
    dst2 = jnp.stack([jnp.concatenate([ei_t[1], padj]),
                      jnp.concatenate([ei_c[1], padj])]
                     ).reshape(2, 1, E_PAD // 16, 16)

    x_pad = jnp.pad(x, ((0, N_PAD - N), (0, 0)))

    hist = _hist(dst2)
    hist_flat = hist.reshape(2, NS, N_PAD).transpose(0, 2, 1)
    xw_t, xw_c = _matmul(x_pad, W_tree, W_cooccurs)
    y_t, y_c = _prescale(xw_t, xw_c, hist_flat)
    acc = _edge_pass(y_t, y_c, src2, dst2)
    out = _final(acc, y_t, y_c, hist_flat,
                 b_tree.reshape(1, D), b_cooccurs.reshape(1, D))
    return out[:N]
